# baseline (speedup 1.0000x reference)
import functools
import math

import jax
import jax.numpy as jnp
from jax import lax
from jax.experimental import pallas as pl
from jax.experimental.pallas import tpu as pltpu
from jax.experimental.pallas import tpu_sc as plsc

_BN = 2000
_CHUNK = 128


def _sc_gather(table, idx2d):
    n_chunks, chunk = idx2d.shape
    d = table.shape[1]
    info = plsc.get_sparse_core_info()
    nc, ns = info.num_cores, info.num_subcores
    nw = nc * ns
    per_w = n_chunks // nw
    rows_per_w = per_w * chunk

    mesh = plsc.VectorSubcoreMesh(core_axis_name="c", subcore_axis_name="s")

    @functools.partial(
        pl.kernel,
        mesh=mesh,
        out_type=jax.ShapeDtypeStruct((n_chunks * chunk, d), jnp.float32),
        scratch_types=[
            pltpu.VMEM((per_w, chunk), jnp.int32),
            pltpu.VMEM((rows_per_w, d), jnp.float32),
            pltpu.SemaphoreType.DMA,
        ],
        compiler_params=pltpu.CompilerParams(use_tc_tiling_on_sc=False),
    )
    def gather_kernel(table_hbm, idx_hbm, out_hbm, idx_v, rows_v, sem):
        wid = lax.axis_index("s") * nc + lax.axis_index("c")
        pltpu.sync_copy(idx_hbm.at[pl.ds(wid * per_w, per_w)], idx_v)
        copies = [
            pltpu.async_copy(
                table_hbm.at[idx_v.at[j]],
                rows_v.at[pl.ds(j * chunk, chunk)],
                sem,
            )
            for j in range(per_w)
        ]
        for cp in copies:
            cp.wait()
        pltpu.sync_copy(rows_v, out_hbm.at[pl.ds(wid * rows_per_w, rows_per_w)])

    return gather_kernel(table, idx2d)


def _mlp_body(emb_ref, w1_ref, b1_ref, w2_ref, b2_ref, out_ref, h_ref):
    @pl.when(pl.program_id(0) == 0)
    def _():
        h = jnp.dot(emb_ref[...], w1_ref[...],
                    preferred_element_type=jnp.float32) + b1_ref[...]
        h_ref[...] = 0.5 * h * (1.0 + lax.erf(h * (1.0 / math.sqrt(2.0))))
    out_ref[...] = (jnp.dot(h_ref[...], w2_ref[:, 0, 0, :],
                            preferred_element_type=jnp.float32)
                    + b2_ref[0])[:, None, None, :]


def kernel(x, emb_table, W1, b1, W2, b2):
    batch, ctx = x.shape
    emb_dim = emb_table.shape[1]
    hid = W1.shape[1]
    vocab = W2.shape[1]
    in_dim = ctx * emb_dim
    grid = vocab // _BN

    idx2d = x.astype(jnp.int32).reshape(-1, _CHUNK)
    rows = _sc_gather(emb_table, idx2d)
    emb_flat = rows.reshape(batch, in_dim)

    out = pl.pallas_call(
        _mlp_body,
        grid=(grid,),
        in_specs=[
            pl.BlockSpec((batch, in_dim), lambda j: (0, 0)),
            pl.BlockSpec((in_dim, hid), lambda j: (0, 0)),
            pl.BlockSpec((1, hid), lambda j: (0, 0)),
            pl.BlockSpec((hid, 1, 1, _BN), lambda j: (0, j, 0, 0)),
            pl.BlockSpec((1, 1, _BN), lambda j: (j, 0, 0)),
        ],
        out_specs=pl.BlockSpec((batch, 1, 1, _BN), lambda j: (0, j, 0, 0)),
        out_shape=jax.ShapeDtypeStruct((batch, grid, 1, _BN), jnp.float32),
        scratch_shapes=[pltpu.VMEM((batch, hid), jnp.float32)],
        compiler_params=pltpu.CompilerParams(
            vmem_limit_bytes=120 * 1024 * 1024),
    )(emb_flat, W1, b1.reshape(1, hid),
      W2.reshape(hid, grid, 1, _BN), b2.reshape(grid, 1, _BN))
    return out.reshape(batch, vocab)


# speedup vs baseline: 2.0030x; 2.0030x over previous
import functools
import math

import jax
import jax.numpy as jnp
from jax import lax
from jax.experimental import pallas as pl
from jax.experimental.pallas import tpu as pltpu
from jax.experimental.pallas import tpu_sc as plsc

_BN = 2048
_NBUF = 4
_CHUNK = 128


def _sc_gather(table, idx2d):
    n_chunks, chunk = idx2d.shape
    d = table.shape[1]
    info = plsc.get_sparse_core_info()
    nc, ns = info.num_cores, info.num_subcores
    nw = nc * ns
    per_w = n_chunks // nw
    rows_per_w = per_w * chunk

    mesh = plsc.VectorSubcoreMesh(core_axis_name="c", subcore_axis_name="s")

    @functools.partial(
        pl.kernel,
        mesh=mesh,
        out_type=jax.ShapeDtypeStruct((n_chunks * chunk, d), jnp.float32),
        scratch_types=[
            pltpu.VMEM((per_w, chunk), jnp.int32),
            pltpu.VMEM((rows_per_w, d), jnp.float32),
            pltpu.SemaphoreType.DMA,
        ],
        compiler_params=pltpu.CompilerParams(use_tc_tiling_on_sc=False),
    )
    def gather_kernel(table_hbm, idx_hbm, out_hbm, idx_v, rows_v, sem):
        wid = lax.axis_index("s") * nc + lax.axis_index("c")
        pltpu.sync_copy(idx_hbm.at[pl.ds(wid * per_w, per_w)], idx_v)
        copies = [
            pltpu.async_copy(
                table_hbm.at[idx_v.at[j]],
                rows_v.at[pl.ds(j * chunk, chunk)],
                sem,
            )
            for j in range(per_w)
        ]
        for cp in copies:
            cp.wait()
        pltpu.sync_copy(rows_v, out_hbm.at[pl.ds(wid * rows_per_w, rows_per_w)])

    return gather_kernel(table, idx2d)


def _gelu(h):
    return 0.5 * h * (1.0 + lax.erf(h * (1.0 / math.sqrt(2.0))))


def _make_mlp_body(nsteps):
    def _copy(buf_ref, out_ref, sems, jj, s):
        return pltpu.make_async_copy(
            buf_ref.at[s],
            out_ref.at[:, pl.ds(jj * _BN, _BN)],
            sems.at[s],
        )

    def _mlp_body(emb_ref, w1_ref, b1_ref, w2_ref, b2_ref, out_ref,
                  h_ref, buf_ref, sems):
        j = pl.program_id(0)
        slot = lax.rem(j, _NBUF)

        @pl.when(j == 0)
        def _():
            h = jnp.dot(emb_ref[...], w1_ref[...],
                        preferred_element_type=jnp.float32) + b1_ref[...]
            h_ref[...] = _gelu(h).astype(jnp.bfloat16)

        @pl.when(j >= _NBUF)
        def _():
            _copy(buf_ref, out_ref, sems, j - _NBUF, slot).wait()

        w2bf = w2_ref[...].astype(jnp.bfloat16)
        buf_ref[slot] = jnp.dot(h_ref[...], w2bf,
                                preferred_element_type=jnp.float32) + b2_ref[...]
        _copy(buf_ref, out_ref, sems, j, slot).start()

        @pl.when(j == nsteps - 1)
        def _():
            for jj in range(max(nsteps - _NBUF, 0), nsteps):
                _copy(buf_ref, out_ref, sems, jj, jj % _NBUF).wait()

    return _mlp_body


def _tail_body(emb_ref, w1_ref, b1_ref, w2_ref, b2_ref, outin_ref, out_ref):
    h = jnp.dot(emb_ref[...], w1_ref[...],
                preferred_element_type=jnp.float32) + b1_ref[...]
    h = _gelu(h).astype(jnp.bfloat16)
    out_ref[...] = jnp.dot(h, w2_ref[...].astype(jnp.bfloat16),
                           preferred_element_type=jnp.float32) + b2_ref[...]


def kernel(x, emb_table, W1, b1, W2, b2):
    batch, ctx = x.shape
    emb_dim = emb_table.shape[1]
    hid = W1.shape[1]
    vocab = W2.shape[1]
    in_dim = ctx * emb_dim
    nfull = vocab // _BN
    tail = vocab - nfull * _BN

    idx2d = x.astype(jnp.int32).reshape(-1, _CHUNK)
    rows = _sc_gather(emb_table, idx2d)
    emb_flat = rows.reshape(batch, in_dim)
    b1r = b1.reshape(1, hid)

    out = pl.pallas_call(
        _make_mlp_body(nfull),
        grid=(nfull,),
        in_specs=[
            pl.BlockSpec((batch, in_dim), lambda j: (0, 0)),
            pl.BlockSpec((in_dim, hid), lambda j: (0, 0)),
            pl.BlockSpec((1, hid), lambda j: (0, 0)),
            pl.BlockSpec((hid, _BN), lambda j: (0, j)),
            pl.BlockSpec((1, _BN), lambda j: (0, j)),
        ],
        out_specs=pl.BlockSpec(memory_space=pl.ANY),
        out_shape=jax.ShapeDtypeStruct((batch, vocab), jnp.float32),
        scratch_shapes=[
            pltpu.VMEM((batch, hid), jnp.bfloat16),
            pltpu.VMEM((_NBUF, batch, _BN), jnp.float32),
            pltpu.SemaphoreType.DMA((_NBUF,)),
        ],
        compiler_params=pltpu.CompilerParams(
            vmem_limit_bytes=120 * 1024 * 1024),
    )(emb_flat, W1, b1r, W2, b2.reshape(1, vocab))

    if tail:
        w2_tail = jnp.pad(lax.slice(W2, (0, nfull * _BN), (hid, vocab)),
                          ((0, 0), (0, _BN - tail)))
        b2_tail = jnp.pad(lax.slice(b2, (nfull * _BN,), (vocab,)),
                          (0, _BN - tail)).reshape(1, _BN)
        out = pl.pallas_call(
            _tail_body,
            grid=(1,),
            in_specs=[
                pl.BlockSpec((batch, in_dim), lambda j: (0, 0)),
                pl.BlockSpec((in_dim, hid), lambda j: (0, 0)),
                pl.BlockSpec((1, hid), lambda j: (0, 0)),
                pl.BlockSpec((hid, _BN), lambda j: (0, 0)),
                pl.BlockSpec((1, _BN), lambda j: (0, 0)),
                pl.BlockSpec(memory_space=pl.ANY),
            ],
            out_specs=pl.BlockSpec((batch, _BN), lambda j: (0, nfull)),
            out_shape=jax.ShapeDtypeStruct((batch, vocab), jnp.float32),
            input_output_aliases={5: 0},
        )(emb_flat, W1, b1r, w2_tail, b2_tail, out)
    return out


# speedup vs baseline: 2.0039x; 1.0005x over previous
import functools
import math

import jax
import jax.numpy as jnp
from jax import lax
from jax.experimental import pallas as pl
from jax.experimental.pallas import tpu as pltpu
from jax.experimental.pallas import tpu_sc as plsc

_BN = 2048
_NBUF = 4
_CHUNK = 128


def _sc_gather(table, idx2d):
    n_chunks, chunk = idx2d.shape
    d = table.shape[1]
    info = plsc.get_sparse_core_info()
    nc, ns = info.num_cores, info.num_subcores
    nw = nc * ns
    per_w = n_chunks // nw
    rows_per_w = per_w * chunk

    mesh = plsc.VectorSubcoreMesh(core_axis_name="c", subcore_axis_name="s")

    @functools.partial(
        pl.kernel,
        mesh=mesh,
        out_type=jax.ShapeDtypeStruct((n_chunks * chunk, d), jnp.float32),
        scratch_types=[
            pltpu.VMEM((per_w, chunk), jnp.int32),
            pltpu.VMEM((rows_per_w, d), jnp.float32),
            pltpu.SemaphoreType.DMA,
        ],
        compiler_params=pltpu.CompilerParams(use_tc_tiling_on_sc=False),
    )
    def gather_kernel(table_hbm, idx_hbm, out_hbm, idx_v, rows_v, sem):
        wid = lax.axis_index("s") * nc + lax.axis_index("c")
        pltpu.sync_copy(idx_hbm.at[pl.ds(wid * per_w, per_w)], idx_v)
        copies = [
            pltpu.async_copy(
                table_hbm.at[idx_v.at[j]],
                rows_v.at[pl.ds(j * chunk, chunk)],
                sem,
            )
            for j in range(per_w)
        ]
        for cp in copies:
            cp.wait()
        pltpu.sync_copy(rows_v, out_hbm.at[pl.ds(wid * rows_per_w, rows_per_w)])

    return gather_kernel(table, idx2d)


def _gelu(h):
    return 0.5 * h * (1.0 + lax.erf(h * (1.0 / math.sqrt(2.0))))


def _make_mlp_body(nsteps):
    def _copy(buf_ref, out_ref, sems, jj, s):
        return pltpu.make_async_copy(
            buf_ref.at[s],
            out_ref.at[:, pl.ds(jj * _BN, _BN)],
            sems.at[s],
        )

    def _mlp_body(emb_ref, w1_ref, b1_ref, w2_ref, b2_ref, out_ref,
                  h_ref, buf_ref, sems):
        j = pl.program_id(0)
        slot = lax.rem(j, _NBUF)

        @pl.when(j == 0)
        def _():
            h = jnp.dot(emb_ref[...], w1_ref[...],
                        preferred_element_type=jnp.float32) + b1_ref[...]
            h_ref[...] = _gelu(h)

        @pl.when(j >= _NBUF)
        def _():
            _copy(buf_ref, out_ref, sems, j - _NBUF, slot).wait()

        buf_ref[slot] = jnp.dot(h_ref[...], w2_ref[...],
                                preferred_element_type=jnp.float32) + b2_ref[...]
        _copy(buf_ref, out_ref, sems, j, slot).start()

        @pl.when(j == nsteps - 1)
        def _():
            for jj in range(max(nsteps - _NBUF, 0), nsteps):
                _copy(buf_ref, out_ref, sems, jj, jj % _NBUF).wait()

    return _mlp_body


def _tail_body(emb_ref, w1_ref, b1_ref, w2_ref, b2_ref, outin_ref, out_ref):
    h = jnp.dot(emb_ref[...], w1_ref[...],
                preferred_element_type=jnp.float32) + b1_ref[...]
    h = _gelu(h)
    out_ref[...] = jnp.dot(h, w2_ref[...],
                           preferred_element_type=jnp.float32) + b2_ref[...]


def kernel(x, emb_table, W1, b1, W2, b2):
    batch, ctx = x.shape
    emb_dim = emb_table.shape[1]
    hid = W1.shape[1]
    vocab = W2.shape[1]
    in_dim = ctx * emb_dim
    nfull = vocab // _BN
    tail = vocab - nfull * _BN

    idx2d = x.astype(jnp.int32).reshape(-1, _CHUNK)
    rows = _sc_gather(emb_table, idx2d)
    emb_flat = rows.reshape(batch, in_dim)
    b1r = b1.reshape(1, hid)

    out = pl.pallas_call(
        _make_mlp_body(nfull),
        grid=(nfull,),
        in_specs=[
            pl.BlockSpec((batch, in_dim), lambda j: (0, 0)),
            pl.BlockSpec((in_dim, hid), lambda j: (0, 0)),
            pl.BlockSpec((1, hid), lambda j: (0, 0)),
            pl.BlockSpec((hid, _BN), lambda j: (0, j)),
            pl.BlockSpec((1, _BN), lambda j: (0, j)),
        ],
        out_specs=pl.BlockSpec(memory_space=pl.ANY),
        out_shape=jax.ShapeDtypeStruct((batch, vocab), jnp.float32),
        scratch_shapes=[
            pltpu.VMEM((batch, hid), jnp.float32),
            pltpu.VMEM((_NBUF, batch, _BN), jnp.float32),
            pltpu.SemaphoreType.DMA((_NBUF,)),
        ],
        compiler_params=pltpu.CompilerParams(
            vmem_limit_bytes=120 * 1024 * 1024),
    )(emb_flat, W1, b1r, W2, b2.reshape(1, vocab))

    if tail:
        w2_tail = jnp.pad(lax.slice(W2, (0, nfull * _BN), (hid, vocab)),
                          ((0, 0), (0, _BN - tail)))
        b2_tail = jnp.pad(lax.slice(b2, (nfull * _BN,), (vocab,)),
                          (0, _BN - tail)).reshape(1, _BN)
        out = pl.pallas_call(
            _tail_body,
            grid=(1,),
            in_specs=[
                pl.BlockSpec((batch, in_dim), lambda j: (0, 0)),
                pl.BlockSpec((in_dim, hid), lambda j: (0, 0)),
                pl.BlockSpec((1, hid), lambda j: (0, 0)),
                pl.BlockSpec((hid, _BN), lambda j: (0, 0)),
                pl.BlockSpec((1, _BN), lambda j: (0, 0)),
                pl.BlockSpec(memory_space=pl.ANY),
            ],
            out_specs=pl.BlockSpec((batch, _BN), lambda j: (0, nfull)),
            out_shape=jax.ShapeDtypeStruct((batch, vocab), jnp.float32),
            input_output_aliases={5: 0},
        )(emb_flat, W1, b1r, w2_tail, b2_tail, out)
    return out


# speedup vs baseline: 2.2523x; 1.1239x over previous
import functools
import math

import jax
import jax.numpy as jnp
from jax import lax
from jax.experimental import pallas as pl
from jax.experimental.pallas import tpu as pltpu
from jax.experimental.pallas import tpu_sc as plsc

_BN = 2048
_NBUF = 4
_NSPLIT = 4
_CHUNK = 128


def _sc_gather(table, idx2d):
    n_chunks, chunk = idx2d.shape
    d = table.shape[1]
    info = plsc.get_sparse_core_info()
    nc, ns = info.num_cores, info.num_subcores
    nw = nc * ns
    per_w = n_chunks // nw
    rows_per_w = per_w * chunk

    mesh = plsc.VectorSubcoreMesh(core_axis_name="c", subcore_axis_name="s")

    @functools.partial(
        pl.kernel,
        mesh=mesh,
        out_type=jax.ShapeDtypeStruct((n_chunks * chunk, d), jnp.float32),
        scratch_types=[
            pltpu.VMEM((per_w, chunk), jnp.int32),
            pltpu.VMEM((rows_per_w, d), jnp.float32),
            pltpu.SemaphoreType.DMA,
        ],
        compiler_params=pltpu.CompilerParams(use_tc_tiling_on_sc=False),
    )
    def gather_kernel(table_hbm, idx_hbm, out_hbm, idx_v, rows_v, sem):
        wid = lax.axis_index("s") * nc + lax.axis_index("c")
        pltpu.sync_copy(idx_hbm.at[pl.ds(wid * per_w, per_w)], idx_v)
        copies = [
            pltpu.async_copy(
                table_hbm.at[idx_v.at[j]],
                rows_v.at[pl.ds(j * chunk, chunk)],
                sem,
            )
            for j in range(per_w)
        ]
        for cp in copies:
            cp.wait()
        pltpu.sync_copy(rows_v, out_hbm.at[pl.ds(wid * rows_per_w, rows_per_w)])

    return gather_kernel(table, idx2d)


def _gelu(h):
    return 0.5 * h * (1.0 + lax.erf(h * (1.0 / math.sqrt(2.0))))


def _make_mlp_body(batch, nsteps, last_w):
    rows = batch // _NSPLIT

    def _copies(buf_ref, out_ref, sems, jj, s, width):
        return [
            pltpu.make_async_copy(
                buf_ref.at[s, pl.ds(r * rows, rows), pl.ds(0, width)],
                out_ref.at[pl.ds(r * rows, rows), pl.ds(jj * _BN, width)],
                sems.at[s, r],
            )
            for r in range(_NSPLIT)
        ]

    def _start(*a):
        for cp in _copies(*a):
            cp.start()

    def _wait(*a):
        for cp in _copies(*a):
            cp.wait()

    def _mlp_body(emb_ref, w1_ref, b1_ref, w2_ref, b2_ref,
                  out_ref, hout_ref, buf_ref, sems):
        j = pl.program_id(0)
        slot = lax.rem(j, _NBUF)

        @pl.when(j == 0)
        def _():
            h = jnp.dot(emb_ref[...], w1_ref[...],
                        preferred_element_type=jnp.float32) + b1_ref[...]
            hout_ref[...] = _gelu(h)

        @pl.when(j >= _NBUF)
        def _():
            _wait(buf_ref, out_ref, sems, j - _NBUF, slot, _BN)

        buf_ref[slot] = jnp.dot(hout_ref[...], w2_ref[...],
                                preferred_element_type=jnp.float32) + b2_ref[...]

        @pl.when(j < nsteps - 1)
        def _():
            _start(buf_ref, out_ref, sems, j, slot, _BN)

        @pl.when(j == nsteps - 1)
        def _():
            _start(buf_ref, out_ref, sems, nsteps - 1,
                   (nsteps - 1) % _NBUF, last_w)
            for jj in range(max(nsteps - _NBUF, 0), nsteps):
                width = last_w if jj == nsteps - 1 else _BN
                _wait(buf_ref, out_ref, sems, jj, jj % _NBUF, width)

    return _mlp_body


def kernel(x, emb_table, W1, b1, W2, b2):
    batch, ctx = x.shape
    emb_dim = emb_table.shape[1]
    hid = W1.shape[1]
    vocab = W2.shape[1]
    in_dim = ctx * emb_dim

    grid = pl.cdiv(vocab, _BN)
    aligned = 128 * ((vocab - (grid - 1) * _BN) // 128)
    covered = (grid - 1) * _BN + aligned

    idx2d = x.astype(jnp.int32).reshape(-1, _CHUNK)
    rows = _sc_gather(emb_table, idx2d)
    emb_flat = rows.reshape(batch, in_dim)

    out, h = pl.pallas_call(
        _make_mlp_body(batch, grid, aligned),
        grid=(grid,),
        in_specs=[
            pl.BlockSpec((batch, in_dim), lambda j: (0, 0)),
            pl.BlockSpec((in_dim, hid), lambda j: (0, 0)),
            pl.BlockSpec((1, hid), lambda j: (0, 0)),
            pl.BlockSpec((hid, _BN), lambda j: (0, j)),
            pl.BlockSpec((1, _BN), lambda j: (0, j)),
        ],
        out_specs=[
            pl.BlockSpec(memory_space=pl.ANY),
            pl.BlockSpec((batch, hid), lambda j: (0, 0)),
        ],
        out_shape=[
            jax.ShapeDtypeStruct((batch, vocab), jnp.float32),
            jax.ShapeDtypeStruct((batch, hid), jnp.float32),
        ],
        scratch_shapes=[
            pltpu.VMEM((_NBUF, batch, _BN), jnp.float32),
            pltpu.SemaphoreType.DMA((_NBUF, _NSPLIT)),
        ],
        compiler_params=pltpu.CompilerParams(
            vmem_limit_bytes=120 * 1024 * 1024),
    )(emb_flat, W1, b1.reshape(1, hid), W2, b2.reshape(1, vocab))

    if covered < vocab:
        sliver = (h @ lax.slice(W2, (0, covered), (hid, vocab))
                  + b2[covered:][None, :])
        out = lax.dynamic_update_slice(out, sliver, (0, covered))
    return out
